# Initial kernel scaffold; baseline (speedup 1.0000x reference)
import jax
import jax.numpy as jnp
from jax.experimental import pallas as pl


def kernel(bin_ids, embedding_weight):
    raise NotImplementedError("write your pallas kernel here")


# speedup vs baseline: 3.6289x; 3.6289x over previous
import functools

import jax
import jax.numpy as jnp
from jax import lax
from jax.experimental import pallas as pl
from jax.experimental.pallas import tpu as pltpu
from jax.experimental.pallas import tpu_sc as plsc

D = 128
L = 128
NC, NS = 2, 16
NW = NC * NS
NBUF = 4
LOOKAHEAD = 2


@functools.lru_cache(maxsize=None)
def _make_gather(n_idx):
  per_w = n_idx // NW
  n_chunks = per_w // L
  n_groups = n_chunks // NBUF
  assert n_idx == NW * n_chunks * L and n_chunks % NBUF == 0

  mesh = plsc.VectorSubcoreMesh(
      core_axis_name="c", subcore_axis_name="s",
      num_cores=NC, num_subcores=NS)

  @functools.partial(
      pl.kernel,
      out_type=jax.ShapeDtypeStruct((n_idx, D), jnp.float32),
      mesh=mesh,
      scratch_types=[
          pltpu.VMEM((1, n_chunks, L), jnp.int32),
          pltpu.VMEM((NBUF, L, D), jnp.float32),
          [pltpu.SemaphoreType.DMA] * NBUF,
          [pltpu.SemaphoreType.DMA] * NBUF,
      ],
  )
  def gather_kernel(idx_hbm, table_hbm, out_hbm, idx_v, rows_v, gsems, psems):
    wid = lax.axis_index("s") * NC + lax.axis_index("c")
    base = wid * per_w

    pltpu.sync_copy(idx_hbm.at[pl.ds(wid, 1)], idx_v)

    def fire_gather(j, b):
      pltpu.async_copy(table_hbm.at[idx_v.at[0, j]], rows_v.at[b], gsems[b])

    def wait_gather(b):
      pltpu.make_async_copy(
          table_hbm.at[idx_v.at[0, 0]], rows_v.at[b], gsems[b]).wait()

    def fire_put(j, b):
      pltpu.async_copy(
          rows_v.at[b], out_hbm.at[pl.ds(base + j * L, L)], psems[b])

    def wait_put(b):
      pltpu.make_async_copy(
          rows_v.at[b], out_hbm.at[pl.ds(base, L)], psems[b]).wait()

    for b in range(NBUF):
      fire_gather(b, b)
      if b >= LOOKAHEAD:
        jp = b - LOOKAHEAD
        wait_gather(jp % NBUF)
        fire_put(jp, jp % NBUF)

    @pl.loop(1, n_groups)
    def _steady(g):
      j0 = g * NBUF
      for b in range(NBUF):
        j = j0 + b
        wait_put(b)
        fire_gather(j, b)
        bp = (b - LOOKAHEAD) % NBUF
        wait_gather(bp)
        fire_put(j - LOOKAHEAD, bp)

    for k in range(LOOKAHEAD):
      j = n_chunks - LOOKAHEAD + k
      b = j % NBUF
      wait_gather(b)
      fire_put(j, b)
    for b in range(NBUF):
      wait_put(b)

  return gather_kernel


def kernel(bin_ids, embedding_weight):
  batch, seq = bin_ids.shape
  n_idx = batch * seq
  idx3d = bin_ids.reshape(NW, n_idx // (NW * L), L).astype(jnp.int32)
  out = _make_gather(n_idx)(idx3d, embedding_weight)
  return out.reshape(batch, seq, D)


# speedup vs baseline: 6.2323x; 1.7174x over previous
import functools

import jax
import jax.numpy as jnp
from jax import lax
from jax.experimental import pallas as pl
from jax.experimental.pallas import tpu as pltpu
from jax.experimental.pallas import tpu_sc as plsc

D = 128
NC, NS = 2, 16
NW = NC * NS
NBUF = 4
LOOKAHEAD = 2
IDX_PAD = 8


@functools.lru_cache(maxsize=None)
def _make_gather(batch, seq):
  per_w = batch // NW
  seq_p = ((seq + IDX_PAD - 1) // IDX_PAD) * IDX_PAD
  n_groups = per_w // NBUF
  assert batch == NW * per_w and per_w % NBUF == 0

  mesh = plsc.VectorSubcoreMesh(
      core_axis_name="c", subcore_axis_name="s",
      num_cores=NC, num_subcores=NS)

  @functools.partial(
      pl.kernel,
      out_type=jax.ShapeDtypeStruct((batch, seq, D), jnp.float32),
      mesh=mesh,
      scratch_types=[
          pltpu.VMEM((1, per_w, seq_p), jnp.int32),
          pltpu.VMEM((NBUF, seq, D), jnp.float32),
          [pltpu.SemaphoreType.DMA] * NBUF,
          [pltpu.SemaphoreType.DMA] * NBUF,
      ],
  )
  def gather_kernel(idx_hbm, table_hbm, out_hbm, idx_v, rows_v, gsems, psems):
    wid = lax.axis_index("s") * NC + lax.axis_index("c")
    base = wid * per_w

    pltpu.sync_copy(idx_hbm.at[pl.ds(wid, 1)], idx_v)

    def fire_gather(j, b):
      pltpu.async_copy(
          table_hbm.at[idx_v.at[0, j, pl.ds(0, seq)]], rows_v.at[b], gsems[b])

    def wait_gather(b):
      pltpu.make_async_copy(
          table_hbm.at[idx_v.at[0, 0, pl.ds(0, seq)]], rows_v.at[b],
          gsems[b]).wait()

    def fire_put(j, b):
      pltpu.async_copy(rows_v.at[b], out_hbm.at[base + j], psems[b])

    def wait_put(b):
      pltpu.make_async_copy(rows_v.at[b], out_hbm.at[base], psems[b]).wait()

    for b in range(NBUF):
      fire_gather(b, b)
      if b >= LOOKAHEAD:
        jp = b - LOOKAHEAD
        wait_gather(jp % NBUF)
        fire_put(jp, jp % NBUF)

    @pl.loop(1, n_groups)
    def _steady(g):
      j0 = g * NBUF
      for b in range(NBUF):
        j = j0 + b
        wait_put(b)
        fire_gather(j, b)
        bp = (b - LOOKAHEAD) % NBUF
        wait_gather(bp)
        fire_put(j - LOOKAHEAD, bp)

    for k in range(LOOKAHEAD):
      j = per_w - LOOKAHEAD + k
      b = j % NBUF
      wait_gather(b)
      fire_put(j, b)
    for b in range(NBUF):
      wait_put(b)

  return gather_kernel


def kernel(bin_ids, embedding_weight):
  batch, seq = bin_ids.shape
  seq_p = ((seq + IDX_PAD - 1) // IDX_PAD) * IDX_PAD
  idx = bin_ids.astype(jnp.int32).reshape(NW, batch // NW, seq)
  idx = jnp.pad(idx, ((0, 0), (0, 0), (0, seq_p - seq)))
  return _make_gather(batch, seq)(idx, embedding_weight)


# speedup vs baseline: 11.1842x; 1.7945x over previous
import functools

import jax
import jax.numpy as jnp
from jax import lax
from jax.experimental import pallas as pl
from jax.experimental.pallas import tpu as pltpu
from jax.experimental.pallas import tpu_sc as plsc

D = 128
NC, NS = 2, 16
NW = NC * NS
NBUF = 4
LOOKAHEAD = 2


@functools.lru_cache(maxsize=None)
def _make_gather(batch, seq):
  per_w = batch // NW
  n_groups = seq // NBUF
  assert batch == NW * per_w and seq % NBUF == 0 and per_w % 8 == 0

  mesh = plsc.VectorSubcoreMesh(
      core_axis_name="c", subcore_axis_name="s",
      num_cores=NC, num_subcores=NS)

  @functools.partial(
      pl.kernel,
      out_type=jax.ShapeDtypeStruct((seq, batch, D), jnp.float32),
      mesh=mesh,
      scratch_types=[
          pltpu.VMEM((1, seq, per_w), jnp.int32),
          pltpu.VMEM((NBUF, per_w, D), jnp.float32),
          [pltpu.SemaphoreType.DMA] * NBUF,
          [pltpu.SemaphoreType.DMA] * NBUF,
      ],
  )
  def gather_kernel(idx_hbm, table_hbm, out_hbm, idx_v, rows_v, gsems, psems):
    wid = lax.axis_index("s") * NC + lax.axis_index("c")
    col0 = wid * per_w

    pltpu.sync_copy(idx_hbm.at[pl.ds(wid, 1)], idx_v)

    def fire_gather(j, b):
      pltpu.async_copy(table_hbm.at[idx_v.at[0, j]], rows_v.at[b], gsems[b])

    def wait_gather(b):
      pltpu.make_async_copy(
          table_hbm.at[idx_v.at[0, 0]], rows_v.at[b], gsems[b]).wait()

    def fire_put(j, b):
      pltpu.async_copy(
          rows_v.at[b], out_hbm.at[j, pl.ds(col0, per_w)], psems[b])

    def wait_put(b):
      pltpu.make_async_copy(
          rows_v.at[b], out_hbm.at[0, pl.ds(col0, per_w)], psems[b]).wait()

    for b in range(NBUF):
      fire_gather(b, b)
      if b >= LOOKAHEAD:
        jp = b - LOOKAHEAD
        wait_gather(jp % NBUF)
        fire_put(jp, jp % NBUF)

    @pl.loop(1, n_groups)
    def _steady(g):
      j0 = g * NBUF
      for b in range(NBUF):
        j = j0 + b
        wait_put(b)
        fire_gather(j, b)
        bp = (b - LOOKAHEAD) % NBUF
        wait_gather(bp)
        fire_put(j - LOOKAHEAD, bp)

    for k in range(LOOKAHEAD):
      j = seq - LOOKAHEAD + k
      b = j % NBUF
      wait_gather(b)
      fire_put(j, b)
    for b in range(NBUF):
      wait_put(b)

  return gather_kernel


def kernel(bin_ids, embedding_weight):
  batch, seq = bin_ids.shape
  per_w = batch // NW
  idx = bin_ids.astype(jnp.int32).reshape(NW, per_w, seq).transpose(0, 2, 1)
  out = _make_gather(batch, seq)(idx, embedding_weight)
  return out.transpose(1, 0, 2)


# speedup vs baseline: 11.2647x; 1.0072x over previous
import functools

import jax
import jax.numpy as jnp
from jax import lax
from jax.experimental import pallas as pl
from jax.experimental.pallas import tpu as pltpu
from jax.experimental.pallas import tpu_sc as plsc

D = 128
NC, NS = 2, 16
NW = NC * NS
NBUF = 4
LOOKAHEAD = 2


@functools.lru_cache(maxsize=None)
def _make_gather(batch, seq):
  per_w = batch // NW
  n_groups = seq // NBUF
  assert batch == NW * per_w and seq % NBUF == 0 and per_w % 8 == 0

  mesh = plsc.VectorSubcoreMesh(
      core_axis_name="c", subcore_axis_name="s",
      num_cores=NC, num_subcores=NS)

  @functools.partial(
      pl.kernel,
      out_type=jax.ShapeDtypeStruct((seq, batch, D), jnp.float32),
      mesh=mesh,
      scratch_types=[
          pltpu.VMEM((seq, per_w), jnp.int32),
          pltpu.VMEM((NBUF, per_w, D), jnp.float32),
          [pltpu.SemaphoreType.DMA] * NBUF,
          [pltpu.SemaphoreType.DMA] * NBUF,
      ],
  )
  def gather_kernel(idx_hbm, table_hbm, out_hbm, idx_v, rows_v, gsems, psems):
    wid = lax.axis_index("s") * NC + lax.axis_index("c")
    col0 = wid * per_w

    pltpu.sync_copy(idx_hbm.at[:, pl.ds(col0, per_w)], idx_v)

    def fire_gather(j, b):
      pltpu.async_copy(table_hbm.at[idx_v.at[j]], rows_v.at[b], gsems[b])

    def wait_gather(b):
      pltpu.make_async_copy(
          table_hbm.at[idx_v.at[0]], rows_v.at[b], gsems[b]).wait()

    def fire_put(j, b):
      pltpu.async_copy(
          rows_v.at[b], out_hbm.at[j, pl.ds(col0, per_w)], psems[b])

    def wait_put(b):
      pltpu.make_async_copy(
          rows_v.at[b], out_hbm.at[0, pl.ds(col0, per_w)], psems[b]).wait()

    for b in range(NBUF):
      fire_gather(b, b)
      if b >= LOOKAHEAD:
        jp = b - LOOKAHEAD
        wait_gather(jp % NBUF)
        fire_put(jp, jp % NBUF)

    @pl.loop(1, n_groups)
    def _steady(g):
      j0 = g * NBUF
      for b in range(NBUF):
        j = j0 + b
        wait_put(b)
        fire_gather(j, b)
        bp = (b - LOOKAHEAD) % NBUF
        wait_gather(bp)
        fire_put(j - LOOKAHEAD, bp)

    for k in range(LOOKAHEAD):
      j = seq - LOOKAHEAD + k
      b = j % NBUF
      wait_gather(b)
      fire_put(j, b)
    for b in range(NBUF):
      wait_put(b)

  return gather_kernel


def kernel(bin_ids, embedding_weight):
  batch, seq = bin_ids.shape
  idx = bin_ids.astype(jnp.int32).T
  out = _make_gather(batch, seq)(idx, embedding_weight)
  return out.transpose(1, 0, 2)


# speedup vs baseline: 11.3130x; 1.0043x over previous
import functools

import jax
import jax.numpy as jnp
from jax import lax
from jax.experimental import pallas as pl
from jax.experimental.pallas import tpu as pltpu
from jax.experimental.pallas import tpu_sc as plsc

D = 128
NC, NS = 2, 16
NW = NC * NS
NBUF = 5
LOOKAHEAD = 2


@functools.lru_cache(maxsize=None)
def _make_gather(batch, seq):
  per_w = batch // NW
  n_groups = seq // NBUF
  assert batch == NW * per_w and seq % NBUF == 0 and per_w % 8 == 0

  mesh = plsc.VectorSubcoreMesh(
      core_axis_name="c", subcore_axis_name="s",
      num_cores=NC, num_subcores=NS)

  @functools.partial(
      pl.kernel,
      out_type=jax.ShapeDtypeStruct((seq, batch, D), jnp.float32),
      mesh=mesh,
      scratch_types=[
          pltpu.VMEM((seq, per_w), jnp.int32),
          pltpu.VMEM((NBUF, per_w, D), jnp.float32),
          [pltpu.SemaphoreType.DMA] * NBUF,
          [pltpu.SemaphoreType.DMA] * NBUF,
      ],
  )
  def gather_kernel(idx_hbm, table_hbm, out_hbm, idx_v, rows_v, gsems, psems):
    wid = lax.axis_index("s") * NC + lax.axis_index("c")
    col0 = wid * per_w

    pltpu.sync_copy(idx_hbm.at[:, pl.ds(col0, per_w)], idx_v)

    def fire_gather(j, b):
      pltpu.async_copy(table_hbm.at[idx_v.at[j]], rows_v.at[b], gsems[b])

    def wait_gather(b):
      pltpu.make_async_copy(
          table_hbm.at[idx_v.at[0]], rows_v.at[b], gsems[b]).wait()

    def fire_put(j, b):
      pltpu.async_copy(
          rows_v.at[b], out_hbm.at[j, pl.ds(col0, per_w)], psems[b])

    def wait_put(b):
      pltpu.make_async_copy(
          rows_v.at[b], out_hbm.at[0, pl.ds(col0, per_w)], psems[b]).wait()

    for b in range(NBUF):
      fire_gather(b, b)
      if b >= LOOKAHEAD:
        jp = b - LOOKAHEAD
        wait_gather(jp % NBUF)
        fire_put(jp, jp % NBUF)

    @pl.loop(1, n_groups)
    def _steady(g):
      j0 = g * NBUF
      for b in range(NBUF):
        j = j0 + b
        wait_put(b)
        fire_gather(j, b)
        bp = (b - LOOKAHEAD) % NBUF
        wait_gather(bp)
        fire_put(j - LOOKAHEAD, bp)

    for k in range(LOOKAHEAD):
      j = seq - LOOKAHEAD + k
      b = j % NBUF
      wait_gather(b)
      fire_put(j, b)
    for b in range(NBUF):
      wait_put(b)

  return gather_kernel


def kernel(bin_ids, embedding_weight):
  batch, seq = bin_ids.shape
  idx = bin_ids.astype(jnp.int32).T
  out = _make_gather(batch, seq)(idx, embedding_weight)
  return out.transpose(1, 0, 2)


# speedup vs baseline: 11.3977x; 1.0075x over previous
import functools

import jax
import jax.numpy as jnp
from jax import lax
from jax.experimental import pallas as pl
from jax.experimental.pallas import tpu as pltpu
from jax.experimental.pallas import tpu_sc as plsc

D = 128
NC, NS = 2, 16
NW = NC * NS
NBUF = 8
LOOKAHEAD = 3
SPLIT = 2


@functools.lru_cache(maxsize=None)
def _make_gather(batch, seq):
  per_w = batch // NW
  rows = per_w // SPLIT
  n_chunks = seq * SPLIT
  n_groups = n_chunks // NBUF
  assert batch == NW * per_w and n_chunks % NBUF == 0 and rows % 8 == 0

  mesh = plsc.VectorSubcoreMesh(
      core_axis_name="c", subcore_axis_name="s",
      num_cores=NC, num_subcores=NS)

  @functools.partial(
      pl.kernel,
      out_type=jax.ShapeDtypeStruct((seq, batch, D), jnp.float32),
      mesh=mesh,
      scratch_types=[
          pltpu.VMEM((seq, per_w), jnp.int32),
          pltpu.VMEM((NBUF, rows, D), jnp.float32),
          [pltpu.SemaphoreType.DMA] * NBUF,
          [pltpu.SemaphoreType.DMA] * NBUF,
      ],
  )
  def gather_kernel(idx_hbm, table_hbm, out_hbm, idx_v, rows_v, gsems, psems):
    wid = lax.axis_index("s") * NC + lax.axis_index("c")
    col0 = wid * per_w

    pltpu.sync_copy(idx_hbm.at[:, pl.ds(col0, per_w)], idx_v)

    def fire_gather(j, b):
      s, h = j // SPLIT, j % SPLIT
      pltpu.async_copy(
          table_hbm.at[idx_v.at[s, pl.ds(h * rows, rows)]], rows_v.at[b],
          gsems[b])

    def wait_gather(b):
      pltpu.make_async_copy(
          table_hbm.at[idx_v.at[0, pl.ds(0, rows)]], rows_v.at[b],
          gsems[b]).wait()

    def fire_put(j, b):
      s, h = j // SPLIT, j % SPLIT
      pltpu.async_copy(
          rows_v.at[b], out_hbm.at[s, pl.ds(col0 + h * rows, rows)], psems[b])

    def wait_put(b):
      pltpu.make_async_copy(
          rows_v.at[b], out_hbm.at[0, pl.ds(col0, rows)], psems[b]).wait()

    for b in range(NBUF):
      fire_gather(b, b)
      if b >= LOOKAHEAD:
        jp = b - LOOKAHEAD
        wait_gather(jp % NBUF)
        fire_put(jp, jp % NBUF)

    @pl.loop(1, n_groups)
    def _steady(g):
      j0 = g * NBUF
      for b in range(NBUF):
        j = j0 + b
        wait_put(b)
        fire_gather(j, b)
        bp = (b - LOOKAHEAD) % NBUF
        wait_gather(bp)
        fire_put(j - LOOKAHEAD, bp)

    for k in range(LOOKAHEAD):
      j = n_chunks - LOOKAHEAD + k
      b = j % NBUF
      wait_gather(b)
      fire_put(j, b)
    for b in range(NBUF):
      wait_put(b)

  return gather_kernel


def kernel(bin_ids, embedding_weight):
  batch, seq = bin_ids.shape
  idx = bin_ids.astype(jnp.int32).T
  out = _make_gather(batch, seq)(idx, embedding_weight)
  return out.transpose(1, 0, 2)
